# Initial kernel scaffold; baseline (speedup 1.0000x reference)
import jax
import jax.numpy as jnp
from jax.experimental import pallas as pl


def kernel(x, edge_index, Wq, bq, Wk, bk, Wv, bv, Wa, ba, skip, a_rel, m_rel, p_rel):
    raise NotImplementedError("write your pallas kernel here")


# speedup vs baseline: 3.0289x; 3.0289x over previous
import functools

import jax
import jax.numpy as jnp
from jax import lax
from jax.experimental import pallas as pl
from jax.experimental.pallas import tpu as pltpu
from jax.experimental.pallas import tpu_sc as plsc

H = 8
D = 16
HD = H * D

NC = 2
NS = 16
L = 16


def _proj_body(x_ref, wq_ref, wk_ref, wv_ref, bq_ref, bk_ref, bv_ref,
               arel_ref, mrel_ref, prow_ref, q_ref, k_ref, v_ref):
  x = x_ref[...]
  q_ref[...] = (x @ wq_ref[...] + bq_ref[...]) * prow_ref[...]
  wk_eff = wk_ref[...] @ arel_ref[...]
  k_ref[...] = x @ wk_eff + (bk_ref[...] @ arel_ref[...])
  wv_eff = wv_ref[...] @ mrel_ref[...]
  v_ref[...] = x @ wv_eff + (bv_ref[...] @ mrel_ref[...])


def _project(x, Wq, bq, Wk, bk, Wv, bv, a_blk, m_blk, prow, blk_n):
  n = x.shape[0]
  grid = (n // blk_n,)
  row_spec = pl.BlockSpec((blk_n, HD), lambda i: (i, 0))
  w_spec = pl.BlockSpec((HD, HD), lambda i: (0, 0))
  b_spec = pl.BlockSpec((1, HD), lambda i: (0, 0))
  return pl.pallas_call(
      _proj_body,
      grid=grid,
      in_specs=[row_spec, w_spec, w_spec, w_spec, b_spec, b_spec, b_spec,
                w_spec, w_spec, b_spec],
      out_specs=[row_spec, row_spec, row_spec],
      out_shape=[jax.ShapeDtypeStruct((n, HD), jnp.float32)] * 3,
  )(x, Wq, Wk, Wv, bq.reshape(1, HD), bk.reshape(1, HD), bv.reshape(1, HD),
    a_blk, m_blk, prow)


def _make_edge_kernel(n_pad, e, chunk):
  mesh = plsc.VectorSubcoreMesh(core_axis_name="c", subcore_axis_name="s")
  edges_per_sub = e // (NC * NS)
  n_chunks = edges_per_sub // chunk
  n_groups = chunk // L
  rows_per_sub = n_pad // NS

  @functools.partial(
      pl.kernel,
      mesh=mesh,
      compiler_params=pltpu.CompilerParams(needs_layout_passes=False,
                                           use_tc_tiling_on_sc=False),
      out_type=(
          jax.ShapeDtypeStruct((NC, n_pad, HD), jnp.float32),
          jax.ShapeDtypeStruct((NC, n_pad, H), jnp.float32),
      ),
      scratch_types=[
          pltpu.VMEM_SHARED((n_pad, HD), jnp.float32),
          pltpu.VMEM_SHARED((n_pad, H), jnp.float32),
          pltpu.VMEM((chunk,), jnp.int32),
          pltpu.VMEM((chunk,), jnp.int32),
          pltpu.VMEM((chunk, HD), jnp.float32),
          pltpu.VMEM((chunk, HD), jnp.float32),
          pltpu.VMEM((chunk, HD), jnp.float32),
          pltpu.VMEM((chunk, H), jnp.float32),
          pltpu.SemaphoreType.DMA,
      ],
  )
  def edge_kernel(q_hbm, k_hbm, v_hbm, src_hbm, tgt_hbm, onum_hbm, oden_hbm,
                  numer, denom, sbuf, tbuf, qb, kb, vb, wt, sem):
    cid = lax.axis_index("c")
    sid = lax.axis_index("s")
    zero16 = jnp.zeros((L,), jnp.float32)

    lanes0 = lax.iota(jnp.int32, L)

    def _zrow(r, c):
      for f in range(0, HD, L):
        vb[r, pl.ds(f, L)] = zero16
      return c

    lax.fori_loop(0, chunk, _zrow, 0, unroll=False)

    def _zwt(i, c):
      fl = i * L + lanes0
      plsc.store_scatter(wt, [fl // H, fl % H], zero16)
      return c

    lax.fori_loop(0, chunk * H // L, _zwt, 0, unroll=False)

    row0 = sid * rows_per_sub
    for j in range(rows_per_sub // chunk):
      pltpu.sync_copy(vb, numer.at[pl.ds(row0 + j * chunk, chunk), :])
      pltpu.sync_copy(wt, denom.at[pl.ds(row0 + j * chunk, chunk), :])
    plsc.subcore_barrier()

    ebase = (cid * NS + sid) * (n_chunks * chunk)
    lanes = lax.iota(jnp.int32, L)

    def chunk_body(j, carry):
      base = ebase + j * chunk
      pltpu.sync_copy(src_hbm.at[pl.ds(base, chunk)], sbuf)
      pltpu.sync_copy(tgt_hbm.at[pl.ds(base, chunk)], tbuf)
      cq = pltpu.async_copy(q_hbm.at[tbuf], qb, sem)
      ck = pltpu.async_copy(k_hbm.at[sbuf], kb, sem)
      cv = pltpu.async_copy(v_hbm.at[sbuf], vb, sem)
      cq.wait()
      ck.wait()
      cv.wait()

      def group_body(g, carry):
        rows = g * L + lanes
        for h in range(H):
          acc = zero16
          for d in range(D):
            col = jnp.full((L,), h * D + d, jnp.int32)
            qv = plsc.load_gather(qb, [rows, col])
            kv = plsc.load_gather(kb, [rows, col])
            acc = acc + qv * kv
          w = jnp.exp(acc)
          plsc.store_scatter(wt, [rows, jnp.full((L,), h, jnp.int32)], w)
          for d in range(D):
            col = jnp.full((L,), h * D + d, jnp.int32)
            vv = plsc.load_gather(vb, [rows, col])
            plsc.store_scatter(vb, [rows, col], vv * w)
        return carry

      lax.fori_loop(0, n_groups, group_body, 0, unroll=False)
      pltpu.sync_copy(vb, numer.at[tbuf], add=True)
      pltpu.sync_copy(wt, denom.at[tbuf], add=True)
      return carry

    lax.fori_loop(0, n_chunks, chunk_body, 0, unroll=False)
    plsc.subcore_barrier()

    pltpu.sync_copy(numer.at[pl.ds(row0, rows_per_sub), :],
                    onum_hbm.at[cid, pl.ds(row0, rows_per_sub), :])
    pltpu.sync_copy(denom.at[pl.ds(row0, rows_per_sub), :],
                    oden_hbm.at[cid, pl.ds(row0, rows_per_sub), :])

  return edge_kernel


def _out_body(num_ref, den_ref, x_ref, wa_ref, ba_ref, r8_ref, mix_ref,
              o_ref):
  num = num_ref[0] + num_ref[1]
  den = den_ref[0] + den_ref[1]
  den_rep = den @ r8_ref[...]
  m = num / (den_rep + 1e-16)
  out = m @ wa_ref[...] + ba_ref[...]
  a = mix_ref[0, 0]
  o_ref[...] = a * out + (1.0 - a) * x_ref[...]


def _combine(num2, den2, x, Wa, ba, r8, mix, blk_n):
  n = x.shape[0]
  grid = (n // blk_n,)
  return pl.pallas_call(
      _out_body,
      grid=grid,
      in_specs=[
          pl.BlockSpec((NC, blk_n, HD), lambda i: (0, i, 0)),
          pl.BlockSpec((NC, blk_n, H), lambda i: (0, i, 0)),
          pl.BlockSpec((blk_n, HD), lambda i: (i, 0)),
          pl.BlockSpec((HD, HD), lambda i: (0, 0)),
          pl.BlockSpec((1, HD), lambda i: (0, 0)),
          pl.BlockSpec((H, HD), lambda i: (0, 0)),
          pl.BlockSpec((1, 1), lambda i: (0, 0)),
      ],
      out_specs=pl.BlockSpec((blk_n, HD), lambda i: (i, 0)),
      out_shape=jax.ShapeDtypeStruct((n, HD), jnp.float32),
  )(num2, den2, x, Wa, ba.reshape(1, HD), r8, mix)


def kernel(x, edge_index, Wq, bq, Wk, bk, Wv, bv, Wa, ba, skip, a_rel,
           m_rel, p_rel):
  n = x.shape[0]
  e = edge_index.shape[1]

  eyeh = jnp.eye(H, dtype=jnp.float32)
  a_blk = (eyeh[:, None, :, None] * a_rel[:, :, None, :]).reshape(HD, HD)
  m_blk = (eyeh[:, None, :, None] * m_rel[:, :, None, :]).reshape(HD, HD)
  prow = jnp.repeat(p_rel / jnp.sqrt(jnp.float32(D)), D).reshape(1, HD)
  r8 = (jnp.eye(H, dtype=jnp.float32)[:, :, None] *
        jnp.ones((D,), jnp.float32)).reshape(H, HD)
  mix = jax.nn.sigmoid(skip).reshape(1, 1).astype(jnp.float32)

  q, k, v = _project(x, Wq, bq, Wk, bk, Wv, bv, a_blk, m_blk, prow, 1000)

  chunk = 80
  n_pad = ((n + 127) // 128) * 128
  edge_kernel = _make_edge_kernel(n_pad, e, chunk)
  num2, den2 = edge_kernel(q, k, v, edge_index[0], edge_index[1])

  return _combine(num2, den2, x, Wa, ba, r8, mix, 1000)


# speedup vs baseline: 3.4142x; 1.1272x over previous
import functools

import jax
import jax.numpy as jnp
from jax import lax
from jax.experimental import pallas as pl
from jax.experimental.pallas import tpu as pltpu
from jax.experimental.pallas import tpu_sc as plsc

H = 8
D = 16
HD = H * D
KV = 2 * HD

NC = 2
NS = 16
L = 16

C = 32
BC = 32
MARGIN = 2
IDXN = (BC + MARGIN) * C


def _proj_body(x_ref, wq_ref, wk_ref, wv_ref, bq_ref, bk_ref, bv_ref,
               arel_ref, mrel_ref, prow_ref, q_ref, kv_ref):
  x = x_ref[...]
  q_ref[...] = (x @ wq_ref[...] + bq_ref[...]) * prow_ref[...]
  wk_eff = wk_ref[...] @ arel_ref[...]
  kv_ref[:, :HD] = x @ wk_eff + (bk_ref[...] @ arel_ref[...])
  wv_eff = wv_ref[...] @ mrel_ref[...]
  kv_ref[:, HD:] = x @ wv_eff + (bv_ref[...] @ mrel_ref[...])


def _project(x, Wq, bq, Wk, bk, Wv, bv, a_blk, m_blk, prow, blk_n):
  n = x.shape[0]
  grid = (n // blk_n,)
  row_spec = pl.BlockSpec((blk_n, HD), lambda i: (i, 0))
  w_spec = pl.BlockSpec((HD, HD), lambda i: (0, 0))
  b_spec = pl.BlockSpec((1, HD), lambda i: (0, 0))
  return pl.pallas_call(
      _proj_body,
      grid=grid,
      in_specs=[row_spec, w_spec, w_spec, w_spec, b_spec, b_spec, b_spec,
                w_spec, w_spec, b_spec],
      out_specs=[row_spec, pl.BlockSpec((blk_n, KV), lambda i: (i, 0))],
      out_shape=[jax.ShapeDtypeStruct((n, HD), jnp.float32),
                 jax.ShapeDtypeStruct((n, KV), jnp.float32)],
  )(x, Wq, Wk, Wv, bq.reshape(1, HD), bk.reshape(1, HD), bv.reshape(1, HD),
    a_blk, m_blk, prow)


def _make_edge_kernel(n_pad, e):
  mesh = plsc.VectorSubcoreMesh(core_axis_name="c", subcore_axis_name="s")
  n_chunks = e // C
  nsub = NC * NS
  per_hi = -(-n_chunks // nsub)
  n_low = n_chunks - (per_hi - 1) * nsub
  nb = -(-per_hi // BC)
  rows_per_sub = n_pad // NS

  @functools.partial(
      pl.kernel,
      mesh=mesh,
      compiler_params=pltpu.CompilerParams(needs_layout_passes=False,
                                           use_tc_tiling_on_sc=False),
      out_type=(
          jax.ShapeDtypeStruct((NC, n_pad, HD), jnp.float32),
          jax.ShapeDtypeStruct((NC, n_pad, L), jnp.float32),
      ),
      scratch_types=[
          pltpu.VMEM_SHARED((n_pad, HD), jnp.float32),
          pltpu.VMEM_SHARED((n_pad, L), jnp.float32),
          pltpu.VMEM((IDXN,), jnp.int32),
          pltpu.VMEM((IDXN,), jnp.int32),
          pltpu.VMEM((IDXN,), jnp.int32),
          pltpu.VMEM((IDXN,), jnp.int32),
          pltpu.VMEM((C,), jnp.int32),
          pltpu.VMEM((C,), jnp.int32),
          pltpu.VMEM((C, HD), jnp.float32),
          pltpu.VMEM((C, HD), jnp.float32),
          pltpu.VMEM((C, KV), jnp.float32),
          pltpu.VMEM((C, KV), jnp.float32),
          pltpu.VMEM((C, HD), jnp.float32),
          pltpu.VMEM((C, HD), jnp.float32),
          pltpu.VMEM((C, L), jnp.float32),
          pltpu.VMEM((C, L), jnp.float32),
          pltpu.SemaphoreType.DMA,
          pltpu.SemaphoreType.DMA,
          pltpu.SemaphoreType.DMA,
          pltpu.SemaphoreType.DMA,
          pltpu.SemaphoreType.DMA,
          pltpu.SemaphoreType.DMA,
      ],
  )
  def edge_kernel(q_hbm, kv_hbm, src_hbm, tgt_hbm, onum_hbm, oden_hbm,
                  numer, denom, sidx0, sidx1, tidx0, tidx1, tbuf0, tbuf1,
                  qb0, qb1, kvb0, kvb1, wv0, wv1, wt0, wt1,
                  sq0, sq1, skv0, skv1, ss0, ss1):
    cid = lax.axis_index("c")
    sid = lax.axis_index("s")
    g = cid * NS + sid
    a = g * per_hi - jnp.maximum(g - n_low, 0)
    hi = a + per_hi - jnp.where(g >= n_low, 1, 0)
    sidx = [sidx0, sidx1]
    tidx = [tidx0, tidx1]
    tbuf = [tbuf0, tbuf1]
    qb = [qb0, qb1]
    kvb = [kvb0, kvb1]
    wv = [wv0, wv1]
    wt = [wt0, wt1]
    sq = [sq0, sq1]
    skv = [skv0, skv1]
    ss = [ss0, ss1]
    zero16 = jnp.zeros((L,), jnp.float32)
    lanes = lax.iota(jnp.int32, L)

    def _zrow(r, cr):
      for f in range(0, HD, L):
        wv0[r, pl.ds(f, L)] = zero16
      return cr

    lax.fori_loop(0, C, _zrow, 0, unroll=False)

    def _zwt(i, cr):
      wt0[i, :] = zero16
      wt1[i, :] = zero16
      return cr

    lax.fori_loop(0, C, _zwt, 0, unroll=False)

    row0 = sid * rows_per_sub
    for j in range(rows_per_sub // C):
      pltpu.sync_copy(wv0, numer.at[pl.ds(row0 + j * C, C), :])
      pltpu.sync_copy(wt0, denom.at[pl.ds(row0 + j * C, C), :])
    plsc.subcore_barrier()

    def issue_gathers(b, ib, off):
      pltpu.async_copy(q_hbm.at[tidx[ib].at[pl.ds(off, C)]], qb[b], sq[b])
      pltpu.async_copy(kv_hbm.at[sidx[ib].at[pl.ds(off, C)]], kvb[b], skv[b])

    def wait_gathers(b, ib, off):
      pltpu.make_async_copy(
          q_hbm.at[tidx[ib].at[pl.ds(off, C)]], qb[b], sq[b]).wait()
      pltpu.make_async_copy(
          kv_hbm.at[sidx[ib].at[pl.ds(off, C)]], kvb[b], skv[b]).wait()

    def wait_scatters(b):
      pltpu.make_async_copy(wv[b], numer.at[tbuf[b]], ss[b]).wait()
      pltpu.make_async_copy(wt[b], denom.at[tbuf[b]], ss[b]).wait()

    def compute(b):
      qq, kvv, wvv, wtt = qb[b], kvb[b], wv[b], wt[b]

      def h_body(i, cr):
        h = i % H
        rows = (i // H) * L + lanes
        colbase = h * D
        acc = zero16
        for d in range(D):
          col = jnp.full((L,), colbase + d, jnp.int32)
          acc = acc + (plsc.load_gather(qq, [rows, col]) *
                       plsc.load_gather(kvv, [rows, col]))
        w = jnp.exp(acc)
        plsc.store_scatter(wtt, [rows, jnp.full((L,), h, jnp.int32)], w)
        for d in range(D):
          cv = jnp.full((L,), HD + colbase + d, jnp.int32)
          vv = plsc.load_gather(kvv, [rows, cv])
          plsc.store_scatter(
              wvv, [rows, jnp.full((L,), colbase + d, jnp.int32)], vv * w)
        return cr

      lax.fori_loop(0, (C // L) * H, h_body, 0, unroll=False)

    pltpu.sync_copy(src_hbm.at[pl.ds(a * C, IDXN)], sidx0)
    pltpu.sync_copy(tgt_hbm.at[pl.ds(a * C, IDXN)], tidx0)
    issue_gathers(0, 0, 0)
    issue_gathers(1, 0, C)

    def block_pair(bp, carry):
      for kb2 in range(2):
        ib = kb2
        bk_idx = 2 * bp + kb2
        cb = a + bk_idx * BC

        @pl.when(bk_idx < nb)
        def _():
          @pl.when(bk_idx >= 1)
          def _():
            pltpu.sync_copy(src_hbm.at[pl.ds(cb * C, IDXN)], sidx[ib])
            pltpu.sync_copy(tgt_hbm.at[pl.ds(cb * C, IDXN)], tidx[ib])

          def pair_body(p, carry2):
            for b in range(2):
              c = cb + 2 * p + b
              off = (2 * p + b) * C

              @pl.when(c < hi)
              def _():
                wait_gathers(b, ib, off)

                @pl.when(c >= a + 2)
                def _():
                  wait_scatters(b)

                tb = tbuf[b]
                tb[pl.ds(0, L)] = tidx[ib][pl.ds(off, L)]
                tb[pl.ds(L, L)] = tidx[ib][pl.ds(off + L, L)]
                compute(b)
                pltpu.async_copy(wv[b], numer.at[tb], ss[b], add=True)
                pltpu.async_copy(wt[b], denom.at[tb], ss[b], add=True)

              @pl.when(c + 2 < hi)
              def _():
                issue_gathers(b, ib, off + 2 * C)
            return carry2

          lax.fori_loop(0, BC // 2, pair_body, 0, unroll=False)
      return carry

    lax.fori_loop(0, (nb + 1) // 2, block_pair, 0, unroll=False)

    for b in range(2):
      wait_scatters(b)
    plsc.subcore_barrier()

    pltpu.sync_copy(numer.at[pl.ds(row0, rows_per_sub), :],
                    onum_hbm.at[cid, pl.ds(row0, rows_per_sub), :])
    pltpu.sync_copy(denom.at[pl.ds(row0, rows_per_sub), :],
                    oden_hbm.at[cid, pl.ds(row0, rows_per_sub), :])

  return edge_kernel


def _out_body(num_ref, den_ref, x_ref, wa_ref, ba_ref, r16_ref, mix_ref,
              o_ref):
  num = num_ref[0] + num_ref[1]
  den = den_ref[0] + den_ref[1]
  den_rep = den @ r16_ref[...]
  m = num / (den_rep + 1e-16)
  out = m @ wa_ref[...] + ba_ref[...]
  a = mix_ref[0, 0]
  o_ref[...] = a * out + (1.0 - a) * x_ref[...]


def _combine(num2, den2, x, Wa, ba, r16, mix, blk_n):
  n = x.shape[0]
  grid = (n // blk_n,)
  return pl.pallas_call(
      _out_body,
      grid=grid,
      in_specs=[
          pl.BlockSpec((NC, blk_n, HD), lambda i: (0, i, 0)),
          pl.BlockSpec((NC, blk_n, L), lambda i: (0, i, 0)),
          pl.BlockSpec((blk_n, HD), lambda i: (i, 0)),
          pl.BlockSpec((HD, HD), lambda i: (0, 0)),
          pl.BlockSpec((1, HD), lambda i: (0, 0)),
          pl.BlockSpec((L, HD), lambda i: (0, 0)),
          pl.BlockSpec((1, 1), lambda i: (0, 0)),
      ],
      out_specs=pl.BlockSpec((blk_n, HD), lambda i: (i, 0)),
      out_shape=jax.ShapeDtypeStruct((n, HD), jnp.float32),
  )(num2, den2, x, Wa, ba.reshape(1, HD), r16, mix)


def kernel(x, edge_index, Wq, bq, Wk, bk, Wv, bv, Wa, ba, skip, a_rel,
           m_rel, p_rel):
  n = x.shape[0]
  e = edge_index.shape[1]

  eyeh = jnp.eye(H, dtype=jnp.float32)
  a_blk = (eyeh[:, None, :, None] * a_rel[:, :, None, :]).reshape(HD, HD)
  m_blk = (eyeh[:, None, :, None] * m_rel[:, :, None, :]).reshape(HD, HD)
  prow = jnp.repeat(p_rel / jnp.sqrt(jnp.float32(D)), D).reshape(1, HD)
  r16 = (jnp.eye(L, H, dtype=jnp.float32)[:, :, None] *
         jnp.ones((D,), jnp.float32)).reshape(L, HD)
  mix = jax.nn.sigmoid(skip).reshape(1, 1).astype(jnp.float32)

  q, kv = _project(x, Wq, bq, Wk, bk, Wv, bv, a_blk, m_blk, prow, 1000)

  zpad = jnp.zeros((IDXN,), jnp.int32)
  srcp = jnp.concatenate([edge_index[0], zpad])
  tgtp = jnp.concatenate([edge_index[1], zpad])

  n_pad = -(-n // (NS * C)) * (NS * C)
  edge_kernel = _make_edge_kernel(n_pad, e)
  num2, den2 = edge_kernel(q, kv, srcp, tgtp)

  return _combine(num2, den2, x, Wa, ba, r16, mix, 1000)


# speedup vs baseline: 3.6473x; 1.0683x over previous
import functools

import jax
import jax.numpy as jnp
from jax import lax
from jax.experimental import pallas as pl
from jax.experimental.pallas import tpu as pltpu
from jax.experimental.pallas import tpu_sc as plsc

H = 8
D = 16
HD = H * D
KV = 2 * HD

NC = 2
NS = 16
L = 16

C = 32
BC = 32
MARGIN = 2
IDXN = (BC + MARGIN) * C


def _proj_body(x_ref, wq_ref, wk_ref, wv_ref, bq_ref, bk_ref, bv_ref,
               arel_ref, mrel_ref, prow_ref, q_ref, kv_ref):
  x = x_ref[...]
  q_ref[...] = (x @ wq_ref[...] + bq_ref[...]) * prow_ref[...]
  wk_eff = wk_ref[...] @ arel_ref[...]
  kv_ref[:, :HD] = x @ wk_eff + (bk_ref[...] @ arel_ref[...])
  wv_eff = wv_ref[...] @ mrel_ref[...]
  kv_ref[:, HD:] = x @ wv_eff + (bv_ref[...] @ mrel_ref[...])


def _project(x, Wq, bq, Wk, bk, Wv, bv, a_blk, m_blk, prow, blk_n):
  n = x.shape[0]
  grid = (n // blk_n,)
  row_spec = pl.BlockSpec((blk_n, HD), lambda i: (i, 0))
  w_spec = pl.BlockSpec((HD, HD), lambda i: (0, 0))
  b_spec = pl.BlockSpec((1, HD), lambda i: (0, 0))
  return pl.pallas_call(
      _proj_body,
      grid=grid,
      in_specs=[row_spec, w_spec, w_spec, w_spec, b_spec, b_spec, b_spec,
                w_spec, w_spec, b_spec],
      out_specs=[row_spec, pl.BlockSpec((blk_n, KV), lambda i: (i, 0))],
      out_shape=[jax.ShapeDtypeStruct((n, HD), jnp.float32),
                 jax.ShapeDtypeStruct((n, KV), jnp.float32)],
  )(x, Wq, Wk, Wv, bq.reshape(1, HD), bk.reshape(1, HD), bv.reshape(1, HD),
    a_blk, m_blk, prow)


def _make_edge_kernel(n_pad, e):
  mesh = plsc.VectorSubcoreMesh(core_axis_name="c", subcore_axis_name="s")
  n_chunks = e // C
  nsub = NC * NS
  per_hi = -(-n_chunks // nsub)
  n_low = n_chunks - (per_hi - 1) * nsub
  nb = -(-per_hi // BC)
  rows_per_sub = n_pad // NS

  @functools.partial(
      pl.kernel,
      mesh=mesh,
      compiler_params=pltpu.CompilerParams(needs_layout_passes=False,
                                           use_tc_tiling_on_sc=False),
      out_type=(
          jax.ShapeDtypeStruct((NC, n_pad, HD), jnp.float32),
          jax.ShapeDtypeStruct((NC, n_pad, L), jnp.float32),
      ),
      scratch_types=[
          pltpu.VMEM_SHARED((n_pad, HD), jnp.float32),
          pltpu.VMEM_SHARED((n_pad, L), jnp.float32),
          pltpu.VMEM((IDXN,), jnp.int32),
          pltpu.VMEM((IDXN,), jnp.int32),
          pltpu.VMEM((IDXN,), jnp.int32),
          pltpu.VMEM((IDXN,), jnp.int32),
          pltpu.VMEM((C,), jnp.int32),
          pltpu.VMEM((C,), jnp.int32),
          pltpu.VMEM((C, HD), jnp.float32),
          pltpu.VMEM((C, HD), jnp.float32),
          pltpu.VMEM((C, KV), jnp.float32),
          pltpu.VMEM((C, KV), jnp.float32),
          pltpu.VMEM((C, HD), jnp.float32),
          pltpu.VMEM((C, HD), jnp.float32),
          pltpu.VMEM((C, L), jnp.float32),
          pltpu.VMEM((C, L), jnp.float32),
          pltpu.SemaphoreType.DMA,
          pltpu.SemaphoreType.DMA,
          pltpu.SemaphoreType.DMA,
          pltpu.SemaphoreType.DMA,
          pltpu.SemaphoreType.DMA,
          pltpu.SemaphoreType.DMA,
      ],
  )
  def edge_kernel(q_hbm, kv_hbm, src_hbm, tgt_hbm, onum_hbm, oden_hbm,
                  numer, denom, sidx0, sidx1, tidx0, tidx1, tbuf0, tbuf1,
                  qb0, qb1, kvb0, kvb1, wv0, wv1, wt0, wt1,
                  sq0, sq1, skv0, skv1, ss0, ss1):
    cid = lax.axis_index("c")
    sid = lax.axis_index("s")
    g = cid * NS + sid
    a = g * per_hi - jnp.maximum(g - n_low, 0)
    hi = a + per_hi - jnp.where(g >= n_low, 1, 0)
    sidx = [sidx0, sidx1]
    tidx = [tidx0, tidx1]
    tbuf = [tbuf0, tbuf1]
    qb = [qb0, qb1]
    kvb = [kvb0, kvb1]
    wv = [wv0, wv1]
    wt = [wt0, wt1]
    sq = [sq0, sq1]
    skv = [skv0, skv1]
    ss = [ss0, ss1]
    zero16 = jnp.zeros((L,), jnp.float32)
    lanes = lax.iota(jnp.int32, L)

    def _zrow(r, cr):
      for f in range(0, HD, L):
        wv0[r, pl.ds(f, L)] = zero16
      return cr

    lax.fori_loop(0, C, _zrow, 0, unroll=False)

    def _zwt(i, cr):
      wt0[i, :] = zero16
      wt1[i, :] = zero16
      return cr

    lax.fori_loop(0, C, _zwt, 0, unroll=False)

    row0 = sid * rows_per_sub
    for j in range(rows_per_sub // C):
      pltpu.sync_copy(wv0, numer.at[pl.ds(row0 + j * C, C), :])
      pltpu.sync_copy(wt0, denom.at[pl.ds(row0 + j * C, C), :])
    plsc.subcore_barrier()

    def issue_gathers(b, ib, off):
      pltpu.async_copy(q_hbm.at[tidx[ib].at[pl.ds(off, C)]], qb[b], sq[b])
      pltpu.async_copy(kv_hbm.at[sidx[ib].at[pl.ds(off, C)]], kvb[b], skv[b])

    def wait_gathers(b, ib, off):
      pltpu.make_async_copy(
          q_hbm.at[tidx[ib].at[pl.ds(off, C)]], qb[b], sq[b]).wait()
      pltpu.make_async_copy(
          kv_hbm.at[sidx[ib].at[pl.ds(off, C)]], kvb[b], skv[b]).wait()

    def wait_scatters(b):
      pltpu.make_async_copy(wv[b], numer.at[tbuf[b]], ss[b]).wait()
      pltpu.make_async_copy(wt[b], denom.at[tbuf[b]], ss[b]).wait()

    def compute(b):
      qq, kvv, wvv, wtt = qb[b], kvb[b], wv[b], wt[b]

      def h_body(i, cr):
        h = i % H
        rows = (i // H) * L + lanes
        colbase = h * D
        accs = [zero16, zero16, zero16, zero16]
        for d in range(D):
          col = jnp.full((L,), colbase + d, jnp.int32)
          accs[d % 4] = accs[d % 4] + (plsc.load_gather(qq, [rows, col]) *
                                       plsc.load_gather(kvv, [rows, col]))
        w = jnp.exp((accs[0] + accs[1]) + (accs[2] + accs[3]))
        plsc.store_scatter(wtt, [rows, jnp.full((L,), h, jnp.int32)], w)
        for d in range(D):
          cv = jnp.full((L,), HD + colbase + d, jnp.int32)
          vv = plsc.load_gather(kvv, [rows, cv])
          plsc.store_scatter(
              wvv, [rows, jnp.full((L,), colbase + d, jnp.int32)], vv * w)
        return cr

      lax.fori_loop(0, (C // L) * H, h_body, 0, unroll=False)

    pltpu.sync_copy(src_hbm.at[pl.ds(a * C, IDXN)], sidx0)
    pltpu.sync_copy(tgt_hbm.at[pl.ds(a * C, IDXN)], tidx0)
    issue_gathers(0, 0, 0)
    issue_gathers(1, 0, C)

    def block_pair(bp, carry):
      for kb2 in range(2):
        ib = kb2
        bk_idx = 2 * bp + kb2
        cb = a + bk_idx * BC

        @pl.when(bk_idx < nb)
        def _():
          @pl.when(bk_idx >= 1)
          def _():
            pltpu.sync_copy(src_hbm.at[pl.ds(cb * C, IDXN)], sidx[ib])
            pltpu.sync_copy(tgt_hbm.at[pl.ds(cb * C, IDXN)], tidx[ib])

          def pair_body(p, carry2):
            for b in range(2):
              c = cb + 2 * p + b
              off = (2 * p + b) * C

              @pl.when(c < hi)
              def _():
                wait_gathers(b, ib, off)

                @pl.when(c >= a + 2)
                def _():
                  wait_scatters(b)

                tb = tbuf[b]
                tb[pl.ds(0, L)] = tidx[ib][pl.ds(off, L)]
                tb[pl.ds(L, L)] = tidx[ib][pl.ds(off + L, L)]
                compute(b)
                pltpu.async_copy(wv[b], numer.at[tb], ss[b], add=True)
                pltpu.async_copy(wt[b], denom.at[tb], ss[b], add=True)

              @pl.when(c + 2 < hi)
              def _():
                issue_gathers(b, ib, off + 2 * C)
            return carry2

          lax.fori_loop(0, BC // 2, pair_body, 0, unroll=False)
      return carry

    lax.fori_loop(0, (nb + 1) // 2, block_pair, 0, unroll=False)

    for b in range(2):
      wait_scatters(b)
    plsc.subcore_barrier()

    pltpu.sync_copy(numer.at[pl.ds(row0, rows_per_sub), :],
                    onum_hbm.at[cid, pl.ds(row0, rows_per_sub), :])
    pltpu.sync_copy(denom.at[pl.ds(row0, rows_per_sub), :],
                    oden_hbm.at[cid, pl.ds(row0, rows_per_sub), :])

  return edge_kernel


def _out_body(num_ref, den_ref, x_ref, wa_ref, ba_ref, r16_ref, mix_ref,
              o_ref):
  num = num_ref[0] + num_ref[1]
  den = den_ref[0] + den_ref[1]
  den_rep = den @ r16_ref[...]
  m = num / (den_rep + 1e-16)
  out = m @ wa_ref[...] + ba_ref[...]
  a = mix_ref[0, 0]
  o_ref[...] = a * out + (1.0 - a) * x_ref[...]


def _combine(num2, den2, x, Wa, ba, r16, mix, blk_n):
  n = x.shape[0]
  grid = (n // blk_n,)
  return pl.pallas_call(
      _out_body,
      grid=grid,
      in_specs=[
          pl.BlockSpec((NC, blk_n, HD), lambda i: (0, i, 0)),
          pl.BlockSpec((NC, blk_n, L), lambda i: (0, i, 0)),
          pl.BlockSpec((blk_n, HD), lambda i: (i, 0)),
          pl.BlockSpec((HD, HD), lambda i: (0, 0)),
          pl.BlockSpec((1, HD), lambda i: (0, 0)),
          pl.BlockSpec((L, HD), lambda i: (0, 0)),
          pl.BlockSpec((1, 1), lambda i: (0, 0)),
      ],
      out_specs=pl.BlockSpec((blk_n, HD), lambda i: (i, 0)),
      out_shape=jax.ShapeDtypeStruct((n, HD), jnp.float32),
  )(num2, den2, x, Wa, ba.reshape(1, HD), r16, mix)


def kernel(x, edge_index, Wq, bq, Wk, bk, Wv, bv, Wa, ba, skip, a_rel,
           m_rel, p_rel):
  n = x.shape[0]
  e = edge_index.shape[1]

  eyeh = jnp.eye(H, dtype=jnp.float32)
  a_blk = (eyeh[:, None, :, None] * a_rel[:, :, None, :]).reshape(HD, HD)
  m_blk = (eyeh[:, None, :, None] * m_rel[:, :, None, :]).reshape(HD, HD)
  prow = jnp.repeat(p_rel / jnp.sqrt(jnp.float32(D)), D).reshape(1, HD)
  r16 = (jnp.eye(L, H, dtype=jnp.float32)[:, :, None] *
         jnp.ones((D,), jnp.float32)).reshape(L, HD)
  mix = jax.nn.sigmoid(skip).reshape(1, 1).astype(jnp.float32)

  q, kv = _project(x, Wq, bq, Wk, bk, Wv, bv, a_blk, m_blk, prow, 1000)

  zpad = jnp.zeros((IDXN,), jnp.int32)
  srcp = jnp.concatenate([edge_index[0], zpad])
  tgtp = jnp.concatenate([edge_index[1], zpad])

  n_pad = -(-n // (NS * C)) * (NS * C)
  edge_kernel = _make_edge_kernel(n_pad, e)
  num2, den2 = edge_kernel(q, kv, srcp, tgtp)

  return _combine(num2, den2, x, Wa, ba, r16, mix, 1000)


# speedup vs baseline: 13.0143x; 3.5682x over previous
import functools

import jax
import jax.numpy as jnp
from jax import lax
from jax.experimental import pallas as pl
from jax.experimental.pallas import tpu as pltpu
from jax.experimental.pallas import tpu_sc as plsc

H = 8
D = 16
HD = H * D
KV = 2 * HD

NC = 2
NS = 16
L = 16

C = 32
BC = 32
MARGIN = 2
IDXN = (BC + MARGIN) * C


def _proj_body(x_ref, wq_ref, wk_ref, wv_ref, bq_ref, bk_ref, bv_ref,
               arel_ref, mrel_ref, prow_ref, q_ref, kv_ref):
  x = x_ref[...]
  q_ref[...] = (x @ wq_ref[...] + bq_ref[...]) * prow_ref[...]
  wk_eff = wk_ref[...] @ arel_ref[...]
  kv_ref[:, :HD] = x @ wk_eff + (bk_ref[...] @ arel_ref[...])
  wv_eff = wv_ref[...] @ mrel_ref[...]
  kv_ref[:, HD:] = x @ wv_eff + (bv_ref[...] @ mrel_ref[...])


def _project(x, Wq, bq, Wk, bk, Wv, bv, a_blk, m_blk, prow, blk_n):
  n = x.shape[0]
  grid = (n // blk_n,)
  row_spec = pl.BlockSpec((blk_n, HD), lambda i: (i, 0))
  w_spec = pl.BlockSpec((HD, HD), lambda i: (0, 0))
  b_spec = pl.BlockSpec((1, HD), lambda i: (0, 0))
  return pl.pallas_call(
      _proj_body,
      grid=grid,
      in_specs=[row_spec, w_spec, w_spec, w_spec, b_spec, b_spec, b_spec,
                w_spec, w_spec, b_spec],
      out_specs=[row_spec, pl.BlockSpec((blk_n, KV), lambda i: (i, 0))],
      out_shape=[jax.ShapeDtypeStruct((n, HD), jnp.float32),
                 jax.ShapeDtypeStruct((n, KV), jnp.float32)],
  )(x, Wq, Wk, Wv, bq.reshape(1, HD), bk.reshape(1, HD), bv.reshape(1, HD),
    a_blk, m_blk, prow)


def _make_edge_kernel(n_pad, e):
  mesh = plsc.VectorSubcoreMesh(core_axis_name="c", subcore_axis_name="s")
  n_chunks = e // C
  nsub = NC * NS
  per_hi = -(-n_chunks // nsub)
  n_low = n_chunks - (per_hi - 1) * nsub
  nb = -(-per_hi // BC)
  rows_per_sub = n_pad // NS

  @functools.partial(
      pl.kernel,
      mesh=mesh,
      compiler_params=pltpu.CompilerParams(needs_layout_passes=False,
                                           use_tc_tiling_on_sc=False),
      out_type=(
          jax.ShapeDtypeStruct((NC, n_pad, HD), jnp.float32),
          jax.ShapeDtypeStruct((NC, n_pad, L), jnp.float32),
      ),
      scratch_types=[
          pltpu.VMEM_SHARED((n_pad, HD), jnp.float32),
          pltpu.VMEM_SHARED((n_pad, L), jnp.float32),
          pltpu.VMEM((IDXN,), jnp.int32),
          pltpu.VMEM((IDXN,), jnp.int32),
          pltpu.VMEM((IDXN,), jnp.int32),
          pltpu.VMEM((IDXN,), jnp.int32),
          pltpu.VMEM((C,), jnp.int32),
          pltpu.VMEM((C,), jnp.int32),
          pltpu.VMEM((C, HD), jnp.float32),
          pltpu.VMEM((C, HD), jnp.float32),
          pltpu.VMEM((C, KV), jnp.float32),
          pltpu.VMEM((C, KV), jnp.float32),
          pltpu.VMEM((C, HD), jnp.float32),
          pltpu.VMEM((C, HD), jnp.float32),
          pltpu.VMEM((C, L), jnp.float32),
          pltpu.VMEM((C, L), jnp.float32),
          pltpu.SemaphoreType.DMA,
          pltpu.SemaphoreType.DMA,
          pltpu.SemaphoreType.DMA,
          pltpu.SemaphoreType.DMA,
          pltpu.SemaphoreType.DMA,
          pltpu.SemaphoreType.DMA,
      ],
  )
  def edge_kernel(q_hbm, kv_hbm, src_hbm, tgt_hbm, onum_hbm, oden_hbm,
                  numer, denom, sidx0, sidx1, tidx0, tidx1, tbuf0, tbuf1,
                  qb0, qb1, kvb0, kvb1, wv0, wv1, wt0, wt1,
                  sq0, sq1, skv0, skv1, ss0, ss1):
    cid = lax.axis_index("c")
    sid = lax.axis_index("s")
    g = cid * NS + sid
    a = g * per_hi - jnp.maximum(g - n_low, 0)
    hi = a + per_hi - jnp.where(g >= n_low, 1, 0)
    sidx = [sidx0, sidx1]
    tidx = [tidx0, tidx1]
    tbuf = [tbuf0, tbuf1]
    qb = [qb0, qb1]
    kvb = [kvb0, kvb1]
    wv = [wv0, wv1]
    wt = [wt0, wt1]
    sq = [sq0, sq1]
    skv = [skv0, skv1]
    ss = [ss0, ss1]
    zero16 = jnp.zeros((L,), jnp.float32)
    lanes = lax.iota(jnp.int32, L)

    def _zrow(r, cr):
      for f in range(0, HD, L):
        wv0[r, pl.ds(f, L)] = zero16
      return cr

    lax.fori_loop(0, C, _zrow, 0, unroll=False)

    def _zwt(i, cr):
      wt0[i, :] = zero16
      wt1[i, :] = zero16
      return cr

    lax.fori_loop(0, C, _zwt, 0, unroll=False)

    row0 = sid * rows_per_sub
    for j in range(rows_per_sub // C):
      pltpu.sync_copy(wv0, numer.at[pl.ds(row0 + j * C, C), :])
      pltpu.sync_copy(wt0, denom.at[pl.ds(row0 + j * C, C), :])
    plsc.subcore_barrier()

    def issue_gathers(b, ib, off):
      pltpu.async_copy(q_hbm.at[tidx[ib].at[pl.ds(off, C)]], qb[b], sq[b])
      pltpu.async_copy(kv_hbm.at[sidx[ib].at[pl.ds(off, C)]], kvb[b], skv[b])

    def wait_gathers(b, ib, off):
      pltpu.make_async_copy(
          q_hbm.at[tidx[ib].at[pl.ds(off, C)]], qb[b], sq[b]).wait()
      pltpu.make_async_copy(
          kv_hbm.at[sidx[ib].at[pl.ds(off, C)]], kvb[b], skv[b]).wait()

    def wait_scatters(b):
      pltpu.make_async_copy(wv[b], numer.at[tbuf[b]], ss[b]).wait()
      pltpu.make_async_copy(wt[b], denom.at[tbuf[b]], ss[b]).wait()

    onehot = [(lanes == h).astype(jnp.float32) for h in range(H)]

    def compute(b):
      qq, kvv, wvv, wtt = qb[b], kvb[b], wv[b], wt[b]

      def e_body(c2, cr):
        row = zero16
        for h in range(H):
          s = jnp.sum(qq[c2, pl.ds(h * D, L)] * kvv[c2, pl.ds(h * D, L)])
          row = row + s * onehot[h]
        wrow = jnp.exp(row)
        wtt[c2, :] = wrow
        for h in range(H):
          w_s = wrow[h]
          wvv[c2, pl.ds(h * D, L)] = kvv[c2, pl.ds(HD + h * D, L)] * w_s
        return cr

      lax.fori_loop(0, C, e_body, 0, unroll=False)

    pltpu.sync_copy(src_hbm.at[pl.ds(a * C, IDXN)], sidx0)
    pltpu.sync_copy(tgt_hbm.at[pl.ds(a * C, IDXN)], tidx0)
    issue_gathers(0, 0, 0)
    issue_gathers(1, 0, C)

    def block_pair(bp, carry):
      for kb2 in range(2):
        ib = kb2
        bk_idx = 2 * bp + kb2
        cb = a + bk_idx * BC

        @pl.when(bk_idx < nb)
        def _():
          @pl.when(bk_idx >= 1)
          def _():
            pltpu.sync_copy(src_hbm.at[pl.ds(cb * C, IDXN)], sidx[ib])
            pltpu.sync_copy(tgt_hbm.at[pl.ds(cb * C, IDXN)], tidx[ib])

          def pair_body(p, carry2):
            for b in range(2):
              c = cb + 2 * p + b
              off = (2 * p + b) * C

              @pl.when(c < hi)
              def _():
                wait_gathers(b, ib, off)

                @pl.when(c >= a + 2)
                def _():
                  wait_scatters(b)

                tb = tbuf[b]
                tb[pl.ds(0, L)] = tidx[ib][pl.ds(off, L)]
                tb[pl.ds(L, L)] = tidx[ib][pl.ds(off + L, L)]
                compute(b)
                pltpu.async_copy(wv[b], numer.at[tb], ss[b], add=True)
                pltpu.async_copy(wt[b], denom.at[tb], ss[b], add=True)

              @pl.when(c + 2 < hi)
              def _():
                issue_gathers(b, ib, off + 2 * C)
            return carry2

          lax.fori_loop(0, BC // 2, pair_body, 0, unroll=False)
      return carry

    lax.fori_loop(0, (nb + 1) // 2, block_pair, 0, unroll=False)

    for b in range(2):
      wait_scatters(b)
    plsc.subcore_barrier()

    pltpu.sync_copy(numer.at[pl.ds(row0, rows_per_sub), :],
                    onum_hbm.at[cid, pl.ds(row0, rows_per_sub), :])
    pltpu.sync_copy(denom.at[pl.ds(row0, rows_per_sub), :],
                    oden_hbm.at[cid, pl.ds(row0, rows_per_sub), :])

  return edge_kernel


def _out_body(num_ref, den_ref, x_ref, wa_ref, ba_ref, r16_ref, mix_ref,
              o_ref):
  num = num_ref[0] + num_ref[1]
  den = den_ref[0] + den_ref[1]
  den_rep = den @ r16_ref[...]
  m = num / (den_rep + 1e-16)
  out = m @ wa_ref[...] + ba_ref[...]
  a = mix_ref[0, 0]
  o_ref[...] = a * out + (1.0 - a) * x_ref[...]


def _combine(num2, den2, x, Wa, ba, r16, mix, blk_n):
  n = x.shape[0]
  grid = (n // blk_n,)
  return pl.pallas_call(
      _out_body,
      grid=grid,
      in_specs=[
          pl.BlockSpec((NC, blk_n, HD), lambda i: (0, i, 0)),
          pl.BlockSpec((NC, blk_n, L), lambda i: (0, i, 0)),
          pl.BlockSpec((blk_n, HD), lambda i: (i, 0)),
          pl.BlockSpec((HD, HD), lambda i: (0, 0)),
          pl.BlockSpec((1, HD), lambda i: (0, 0)),
          pl.BlockSpec((L, HD), lambda i: (0, 0)),
          pl.BlockSpec((1, 1), lambda i: (0, 0)),
      ],
      out_specs=pl.BlockSpec((blk_n, HD), lambda i: (i, 0)),
      out_shape=jax.ShapeDtypeStruct((n, HD), jnp.float32),
  )(num2, den2, x, Wa, ba.reshape(1, HD), r16, mix)


def kernel(x, edge_index, Wq, bq, Wk, bk, Wv, bv, Wa, ba, skip, a_rel,
           m_rel, p_rel):
  n = x.shape[0]
  e = edge_index.shape[1]

  eyeh = jnp.eye(H, dtype=jnp.float32)
  a_blk = (eyeh[:, None, :, None] * a_rel[:, :, None, :]).reshape(HD, HD)
  m_blk = (eyeh[:, None, :, None] * m_rel[:, :, None, :]).reshape(HD, HD)
  prow = jnp.repeat(p_rel / jnp.sqrt(jnp.float32(D)), D).reshape(1, HD)
  r16 = (jnp.eye(L, H, dtype=jnp.float32)[:, :, None] *
         jnp.ones((D,), jnp.float32)).reshape(L, HD)
  mix = jax.nn.sigmoid(skip).reshape(1, 1).astype(jnp.float32)

  q, kv = _project(x, Wq, bq, Wk, bk, Wv, bv, a_blk, m_blk, prow, 1000)

  zpad = jnp.zeros((IDXN,), jnp.int32)
  srcp = jnp.concatenate([edge_index[0], zpad])
  tgtp = jnp.concatenate([edge_index[1], zpad])

  n_pad = -(-n // (NS * C)) * (NS * C)
  edge_kernel = _make_edge_kernel(n_pad, e)
  num2, den2 = edge_kernel(q, kv, srcp, tgtp)

  return _combine(num2, den2, x, Wa, ba, r16, mix, 1000)


# speedup vs baseline: 14.8585x; 1.1417x over previous
import functools

import jax
import jax.numpy as jnp
from jax import lax
from jax.experimental import pallas as pl
from jax.experimental.pallas import tpu as pltpu
from jax.experimental.pallas import tpu_sc as plsc

H = 8
D = 16
HD = H * D
KV = 2 * HD

NC = 2
NS = 16
L = 16

C = 32
BC = 32
MARGIN = 2
IDXN = (BC + MARGIN) * C


def _proj_body(x_ref, wq_ref, wk_ref, wv_ref, bq_ref, bk_ref, bv_ref,
               arel_ref, mrel_ref, prow_ref, q_ref, kv_ref):
  x = x_ref[...]
  q_ref[...] = (x @ wq_ref[...] + bq_ref[...]) * prow_ref[...]
  wk_eff = wk_ref[...] @ arel_ref[...]
  kv_ref[:, :HD] = x @ wk_eff + (bk_ref[...] @ arel_ref[...])
  wv_eff = wv_ref[...] @ mrel_ref[...]
  kv_ref[:, HD:] = x @ wv_eff + (bv_ref[...] @ mrel_ref[...])


def _project(x, Wq, bq, Wk, bk, Wv, bv, a_blk, m_blk, prow, blk_n):
  n = x.shape[0]
  grid = (n // blk_n,)
  row_spec = pl.BlockSpec((blk_n, HD), lambda i: (i, 0))
  w_spec = pl.BlockSpec((HD, HD), lambda i: (0, 0))
  b_spec = pl.BlockSpec((1, HD), lambda i: (0, 0))
  return pl.pallas_call(
      _proj_body,
      grid=grid,
      in_specs=[row_spec, w_spec, w_spec, w_spec, b_spec, b_spec, b_spec,
                w_spec, w_spec, b_spec],
      out_specs=[row_spec, pl.BlockSpec((blk_n, KV), lambda i: (i, 0))],
      out_shape=[jax.ShapeDtypeStruct((n, HD), jnp.float32),
                 jax.ShapeDtypeStruct((n, KV), jnp.float32)],
  )(x, Wq, Wk, Wv, bq.reshape(1, HD), bk.reshape(1, HD), bv.reshape(1, HD),
    a_blk, m_blk, prow)


def _make_edge_kernel(n_pad, e):
  mesh = plsc.VectorSubcoreMesh(core_axis_name="c", subcore_axis_name="s")
  n_chunks = e // C
  nsub = NC * NS
  per_hi = -(-n_chunks // nsub)
  n_low = n_chunks - (per_hi - 1) * nsub
  nb = -(-per_hi // BC)
  rows_per_sub = n_pad // NS

  @functools.partial(
      pl.kernel,
      mesh=mesh,
      compiler_params=pltpu.CompilerParams(needs_layout_passes=False,
                                           use_tc_tiling_on_sc=False),
      out_type=(
          jax.ShapeDtypeStruct((NC, n_pad, HD), jnp.float32),
          jax.ShapeDtypeStruct((NC, n_pad, L), jnp.float32),
      ),
      scratch_types=[
          pltpu.VMEM_SHARED((n_pad, HD), jnp.float32),
          pltpu.VMEM_SHARED((n_pad, L), jnp.float32),
          pltpu.VMEM((IDXN,), jnp.int32),
          pltpu.VMEM((IDXN,), jnp.int32),
          pltpu.VMEM((IDXN,), jnp.int32),
          pltpu.VMEM((IDXN,), jnp.int32),
          pltpu.VMEM((C,), jnp.int32),
          pltpu.VMEM((C,), jnp.int32),
          pltpu.VMEM((C, HD), jnp.float32),
          pltpu.VMEM((C, HD), jnp.float32),
          pltpu.VMEM((C, KV), jnp.float32),
          pltpu.VMEM((C, KV), jnp.float32),
          pltpu.VMEM((C, HD), jnp.float32),
          pltpu.VMEM((C, HD), jnp.float32),
          pltpu.VMEM((C, L), jnp.float32),
          pltpu.VMEM((C, L), jnp.float32),
          pltpu.SemaphoreType.DMA,
          pltpu.SemaphoreType.DMA,
          pltpu.SemaphoreType.DMA,
          pltpu.SemaphoreType.DMA,
          pltpu.SemaphoreType.DMA,
          pltpu.SemaphoreType.DMA,
      ],
  )
  def edge_kernel(q_hbm, kv_hbm, src_hbm, tgt_hbm, onum_hbm, oden_hbm,
                  numer, denom, sidx0, sidx1, tidx0, tidx1, tbuf0, tbuf1,
                  qb0, qb1, kvb0, kvb1, wv0, wv1, wt0, wt1,
                  sq0, sq1, skv0, skv1, ss0, ss1):
    cid = lax.axis_index("c")
    sid = lax.axis_index("s")
    g = cid * NS + sid
    a = g * per_hi - jnp.maximum(g - n_low, 0)
    hi = a + per_hi - jnp.where(g >= n_low, 1, 0)
    sidx = [sidx0, sidx1]
    tidx = [tidx0, tidx1]
    tbuf = [tbuf0, tbuf1]
    qb = [qb0, qb1]
    kvb = [kvb0, kvb1]
    wv = [wv0, wv1]
    wt = [wt0, wt1]
    sq = [sq0, sq1]
    skv = [skv0, skv1]
    ss = [ss0, ss1]
    zero16 = jnp.zeros((L,), jnp.float32)
    lanes = lax.iota(jnp.int32, L)

    def _zrow(r, cr):
      for f in range(0, HD, L):
        wv0[r, pl.ds(f, L)] = zero16
      return cr

    lax.fori_loop(0, C, _zrow, 0, unroll=False)

    def _zwt(i, cr):
      wt0[i, :] = zero16
      wt1[i, :] = zero16
      return cr

    lax.fori_loop(0, C, _zwt, 0, unroll=False)

    row0 = sid * rows_per_sub
    for j in range(rows_per_sub // C):
      pltpu.sync_copy(wv0, numer.at[pl.ds(row0 + j * C, C), :])
      pltpu.sync_copy(wt0, denom.at[pl.ds(row0 + j * C, C), :])
    plsc.subcore_barrier()

    def issue_gathers(b, ib, off):
      pltpu.async_copy(q_hbm.at[tidx[ib].at[pl.ds(off, C)]], qb[b], sq[b])
      pltpu.async_copy(kv_hbm.at[sidx[ib].at[pl.ds(off, C)]], kvb[b], skv[b])

    def wait_gathers(b, ib, off):
      pltpu.make_async_copy(
          q_hbm.at[tidx[ib].at[pl.ds(off, C)]], qb[b], sq[b]).wait()
      pltpu.make_async_copy(
          kv_hbm.at[sidx[ib].at[pl.ds(off, C)]], kvb[b], skv[b]).wait()

    def wait_scatters(b):
      pltpu.make_async_copy(wv[b], numer.at[tbuf[b]], ss[b]).wait()
      pltpu.make_async_copy(wt[b], denom.at[tbuf[b]], ss[b]).wait()

    onehot = [(lanes == h).astype(jnp.float32) for h in range(H)]

    def compute(b):
      qq, kvv, wvv, wtt = qb[b], kvb[b], wv[b], wt[b]

      def e_body(i2, cr):
        c0 = i2 * 2
        c1 = c0 + 1
        rows = [zero16, zero16, zero16, zero16]
        for h in range(H):
          s0 = jnp.sum(qq[c0, pl.ds(h * D, L)] * kvv[c0, pl.ds(h * D, L)])
          s1 = jnp.sum(qq[c1, pl.ds(h * D, L)] * kvv[c1, pl.ds(h * D, L)])
          rows[(h % 2) * 2] = rows[(h % 2) * 2] + s0 * onehot[h]
          rows[(h % 2) * 2 + 1] = rows[(h % 2) * 2 + 1] + s1 * onehot[h]
        w0 = jnp.exp(rows[0] + rows[2])
        w1 = jnp.exp(rows[1] + rows[3])
        wtt[c0, :] = w0
        wtt[c1, :] = w1
        for h in range(H):
          wvv[c0, pl.ds(h * D, L)] = kvv[c0, pl.ds(HD + h * D, L)] * w0[h]
          wvv[c1, pl.ds(h * D, L)] = kvv[c1, pl.ds(HD + h * D, L)] * w1[h]
        return cr

      lax.fori_loop(0, C // 2, e_body, 0, unroll=False)

    pltpu.sync_copy(src_hbm.at[pl.ds(a * C, IDXN)], sidx0)
    pltpu.sync_copy(tgt_hbm.at[pl.ds(a * C, IDXN)], tidx0)
    issue_gathers(0, 0, 0)
    issue_gathers(1, 0, C)

    def block_pair(bp, carry):
      for kb2 in range(2):
        ib = kb2
        bk_idx = 2 * bp + kb2
        cb = a + bk_idx * BC

        @pl.when(bk_idx < nb)
        def _():
          @pl.when(bk_idx >= 1)
          def _():
            pltpu.sync_copy(src_hbm.at[pl.ds(cb * C, IDXN)], sidx[ib])
            pltpu.sync_copy(tgt_hbm.at[pl.ds(cb * C, IDXN)], tidx[ib])

          def pair_body(p, carry2):
            for b in range(2):
              c = cb + 2 * p + b
              off = (2 * p + b) * C

              @pl.when(c < hi)
              def _():
                wait_gathers(b, ib, off)

                @pl.when(c >= a + 2)
                def _():
                  wait_scatters(b)

                tb = tbuf[b]
                tb[pl.ds(0, L)] = tidx[ib][pl.ds(off, L)]
                tb[pl.ds(L, L)] = tidx[ib][pl.ds(off + L, L)]
                compute(b)
                pltpu.async_copy(wv[b], numer.at[tb], ss[b], add=True)
                pltpu.async_copy(wt[b], denom.at[tb], ss[b], add=True)

              @pl.when(c + 2 < hi)
              def _():
                issue_gathers(b, ib, off + 2 * C)
            return carry2

          lax.fori_loop(0, BC // 2, pair_body, 0, unroll=False)
      return carry

    lax.fori_loop(0, (nb + 1) // 2, block_pair, 0, unroll=False)

    for b in range(2):
      wait_scatters(b)
    plsc.subcore_barrier()

    pltpu.sync_copy(numer.at[pl.ds(row0, rows_per_sub), :],
                    onum_hbm.at[cid, pl.ds(row0, rows_per_sub), :])
    pltpu.sync_copy(denom.at[pl.ds(row0, rows_per_sub), :],
                    oden_hbm.at[cid, pl.ds(row0, rows_per_sub), :])

  return edge_kernel


def _out_body(num_ref, den_ref, x_ref, wa_ref, ba_ref, r16_ref, mix_ref,
              o_ref):
  num = num_ref[0] + num_ref[1]
  den = den_ref[0] + den_ref[1]
  den_rep = den @ r16_ref[...]
  m = num / (den_rep + 1e-16)
  out = m @ wa_ref[...] + ba_ref[...]
  a = mix_ref[0, 0]
  o_ref[...] = a * out + (1.0 - a) * x_ref[...]


def _combine(num2, den2, x, Wa, ba, r16, mix, blk_n):
  n = x.shape[0]
  grid = (n // blk_n,)
  return pl.pallas_call(
      _out_body,
      grid=grid,
      in_specs=[
          pl.BlockSpec((NC, blk_n, HD), lambda i: (0, i, 0)),
          pl.BlockSpec((NC, blk_n, L), lambda i: (0, i, 0)),
          pl.BlockSpec((blk_n, HD), lambda i: (i, 0)),
          pl.BlockSpec((HD, HD), lambda i: (0, 0)),
          pl.BlockSpec((1, HD), lambda i: (0, 0)),
          pl.BlockSpec((L, HD), lambda i: (0, 0)),
          pl.BlockSpec((1, 1), lambda i: (0, 0)),
      ],
      out_specs=pl.BlockSpec((blk_n, HD), lambda i: (i, 0)),
      out_shape=jax.ShapeDtypeStruct((n, HD), jnp.float32),
  )(num2, den2, x, Wa, ba.reshape(1, HD), r16, mix)


def kernel(x, edge_index, Wq, bq, Wk, bk, Wv, bv, Wa, ba, skip, a_rel,
           m_rel, p_rel):
  n = x.shape[0]
  e = edge_index.shape[1]

  eyeh = jnp.eye(H, dtype=jnp.float32)
  a_blk = (eyeh[:, None, :, None] * a_rel[:, :, None, :]).reshape(HD, HD)
  m_blk = (eyeh[:, None, :, None] * m_rel[:, :, None, :]).reshape(HD, HD)
  prow = jnp.repeat(p_rel / jnp.sqrt(jnp.float32(D)), D).reshape(1, HD)
  r16 = (jnp.eye(L, H, dtype=jnp.float32)[:, :, None] *
         jnp.ones((D,), jnp.float32)).reshape(L, HD)
  mix = jax.nn.sigmoid(skip).reshape(1, 1).astype(jnp.float32)

  q, kv = _project(x, Wq, bq, Wk, bk, Wv, bv, a_blk, m_blk, prow, 1000)

  zpad = jnp.zeros((IDXN,), jnp.int32)
  srcp = jnp.concatenate([edge_index[0], zpad])
  tgtp = jnp.concatenate([edge_index[1], zpad])

  n_pad = -(-n // (NS * C)) * (NS * C)
  edge_kernel = _make_edge_kernel(n_pad, e)
  num2, den2 = edge_kernel(q, kv, srcp, tgtp)

  return _combine(num2, den2, x, Wa, ba, r16, mix, 1000)


# speedup vs baseline: 16.1245x; 1.0852x over previous
import functools

import jax
import jax.numpy as jnp
from jax import lax
from jax.experimental import pallas as pl
from jax.experimental.pallas import tpu as pltpu
from jax.experimental.pallas import tpu_sc as plsc

H = 8
D = 16
HD = H * D
KV = 2 * HD

NC = 2
NS = 16
L = 16

C = 32
BC = 32
MARGIN = 2
IDXN = (BC + MARGIN) * C


def _proj_body(x_ref, wq_ref, wk_ref, wv_ref, bq_ref, bk_ref, bv_ref,
               arel_ref, mrel_ref, prow_ref, q_ref, kv_ref):
  x = x_ref[...]
  q_ref[...] = (x @ wq_ref[...] + bq_ref[...]) * prow_ref[...]
  wk_eff = wk_ref[...] @ arel_ref[...]
  kv_ref[:, :HD] = x @ wk_eff + (bk_ref[...] @ arel_ref[...])
  wv_eff = wv_ref[...] @ mrel_ref[...]
  kv_ref[:, HD:] = x @ wv_eff + (bv_ref[...] @ mrel_ref[...])


def _project(x, Wq, bq, Wk, bk, Wv, bv, a_blk, m_blk, prow, blk_n):
  n = x.shape[0]
  grid = (n // blk_n,)
  row_spec = pl.BlockSpec((blk_n, HD), lambda i: (i, 0))
  w_spec = pl.BlockSpec((HD, HD), lambda i: (0, 0))
  b_spec = pl.BlockSpec((1, HD), lambda i: (0, 0))
  return pl.pallas_call(
      _proj_body,
      grid=grid,
      in_specs=[row_spec, w_spec, w_spec, w_spec, b_spec, b_spec, b_spec,
                w_spec, w_spec, b_spec],
      out_specs=[row_spec, pl.BlockSpec((blk_n, KV), lambda i: (i, 0))],
      out_shape=[jax.ShapeDtypeStruct((n, HD), jnp.float32),
                 jax.ShapeDtypeStruct((n, KV), jnp.float32)],
  )(x, Wq, Wk, Wv, bq.reshape(1, HD), bk.reshape(1, HD), bv.reshape(1, HD),
    a_blk, m_blk, prow)


def _make_edge_kernel(n_pad, e):
  mesh = plsc.VectorSubcoreMesh(core_axis_name="c", subcore_axis_name="s")
  n_chunks = e // C
  nsub = NC * NS
  per_hi = -(-n_chunks // nsub)
  n_low = n_chunks - (per_hi - 1) * nsub
  nb = -(-per_hi // BC)
  rows_per_sub = n_pad // NS

  @functools.partial(
      pl.kernel,
      mesh=mesh,
      compiler_params=pltpu.CompilerParams(needs_layout_passes=False,
                                           use_tc_tiling_on_sc=False),
      out_type=(
          jax.ShapeDtypeStruct((NC, n_pad, HD), jnp.float32),
          jax.ShapeDtypeStruct((NC, n_pad, L), jnp.float32),
      ),
      scratch_types=[
          pltpu.VMEM_SHARED((n_pad, HD), jnp.float32),
          pltpu.VMEM_SHARED((n_pad, L), jnp.float32),
          pltpu.VMEM((IDXN,), jnp.int32),
          pltpu.VMEM((IDXN,), jnp.int32),
          pltpu.VMEM((IDXN,), jnp.int32),
          pltpu.VMEM((IDXN,), jnp.int32),
          pltpu.VMEM((C,), jnp.int32),
          pltpu.VMEM((C,), jnp.int32),
          pltpu.VMEM((C, HD), jnp.float32),
          pltpu.VMEM((C, HD), jnp.float32),
          pltpu.VMEM((C, KV), jnp.float32),
          pltpu.VMEM((C, KV), jnp.float32),
          pltpu.VMEM((C, HD), jnp.float32),
          pltpu.VMEM((C, HD), jnp.float32),
          pltpu.VMEM((C, L), jnp.float32),
          pltpu.VMEM((C, L), jnp.float32),
          pltpu.SemaphoreType.DMA,
          pltpu.SemaphoreType.DMA,
          pltpu.SemaphoreType.DMA,
          pltpu.SemaphoreType.DMA,
          pltpu.SemaphoreType.DMA,
          pltpu.SemaphoreType.DMA,
      ],
  )
  def edge_kernel(q_hbm, kv_hbm, src_hbm, tgt_hbm, onum_hbm, oden_hbm,
                  numer, denom, sidx0, sidx1, tidx0, tidx1, tbuf0, tbuf1,
                  qb0, qb1, kvb0, kvb1, wv0, wv1, wt0, wt1,
                  sq0, sq1, skv0, skv1, ss0, ss1):
    cid = lax.axis_index("c")
    sid = lax.axis_index("s")
    g = cid * NS + sid
    a = g * per_hi - jnp.maximum(g - n_low, 0)
    hi = a + per_hi - jnp.where(g >= n_low, 1, 0)
    sidx = [sidx0, sidx1]
    tidx = [tidx0, tidx1]
    tbuf = [tbuf0, tbuf1]
    qb = [qb0, qb1]
    kvb = [kvb0, kvb1]
    wv = [wv0, wv1]
    wt = [wt0, wt1]
    sq = [sq0, sq1]
    skv = [skv0, skv1]
    ss = [ss0, ss1]
    zero16 = jnp.zeros((L,), jnp.float32)
    lanes = lax.iota(jnp.int32, L)

    def _zrow(r, cr):
      for f in range(0, HD, L):
        wv0[r, pl.ds(f, L)] = zero16
      return cr

    lax.fori_loop(0, C, _zrow, 0, unroll=False)

    def _zwt(i, cr):
      wt0[i, :] = zero16
      wt1[i, :] = zero16
      return cr

    lax.fori_loop(0, C, _zwt, 0, unroll=False)

    row0 = sid * rows_per_sub
    for j in range(rows_per_sub // C):
      pltpu.sync_copy(wv0, numer.at[pl.ds(row0 + j * C, C), :])
      pltpu.sync_copy(wt0, denom.at[pl.ds(row0 + j * C, C), :])
    plsc.subcore_barrier()

    def issue_gathers(b, ib, off):
      pltpu.async_copy(q_hbm.at[tidx[ib].at[pl.ds(off, C)]], qb[b], sq[b])
      pltpu.async_copy(kv_hbm.at[sidx[ib].at[pl.ds(off, C)]], kvb[b], skv[b])

    def wait_gathers(b, ib, off):
      pltpu.make_async_copy(
          q_hbm.at[tidx[ib].at[pl.ds(off, C)]], qb[b], sq[b]).wait()
      pltpu.make_async_copy(
          kv_hbm.at[sidx[ib].at[pl.ds(off, C)]], kvb[b], skv[b]).wait()

    def wait_scatters(b):
      pltpu.make_async_copy(wv[b], numer.at[tbuf[b]], ss[b]).wait()
      pltpu.make_async_copy(wt[b], denom.at[tbuf[b]], ss[b]).wait()

    onehot = [(lanes == h).astype(jnp.float32) for h in range(H)]

    def compute(b):
      qq, kvv, wvv, wtt = qb[b], kvb[b], wv[b], wt[b]

      def e_body(i4, cr):
        cs = [i4 * 4, i4 * 4 + 1, i4 * 4 + 2, i4 * 4 + 3]
        rows = [[zero16, zero16] for _ in range(4)]
        for h in range(H):
          for j, cj in enumerate(cs):
            sj = jnp.sum(qq[cj, pl.ds(h * D, L)] * kvv[cj, pl.ds(h * D, L)])
            rows[j][h % 2] = rows[j][h % 2] + sj * onehot[h]
        ws = [jnp.exp(rows[j][0] + rows[j][1]) for j in range(4)]
        for j, cj in enumerate(cs):
          wtt[cj, :] = ws[j]
        for h in range(H):
          for j, cj in enumerate(cs):
            wvv[cj, pl.ds(h * D, L)] = (
                kvv[cj, pl.ds(HD + h * D, L)] * ws[j][h])
        return cr

      lax.fori_loop(0, C // 4, e_body, 0, unroll=False)

    pltpu.sync_copy(src_hbm.at[pl.ds(a * C, IDXN)], sidx0)
    pltpu.sync_copy(tgt_hbm.at[pl.ds(a * C, IDXN)], tidx0)
    issue_gathers(0, 0, 0)
    issue_gathers(1, 0, C)

    def block_pair(bp, carry):
      for kb2 in range(2):
        ib = kb2
        bk_idx = 2 * bp + kb2
        cb = a + bk_idx * BC

        @pl.when(bk_idx < nb)
        def _():
          @pl.when(bk_idx >= 1)
          def _():
            pltpu.sync_copy(src_hbm.at[pl.ds(cb * C, IDXN)], sidx[ib])
            pltpu.sync_copy(tgt_hbm.at[pl.ds(cb * C, IDXN)], tidx[ib])

          def pair_body(p, carry2):
            for b in range(2):
              c = cb + 2 * p + b
              off = (2 * p + b) * C

              @pl.when(c < hi)
              def _():
                wait_gathers(b, ib, off)

                @pl.when(c >= a + 2)
                def _():
                  wait_scatters(b)

                tb = tbuf[b]
                tb[pl.ds(0, L)] = tidx[ib][pl.ds(off, L)]
                tb[pl.ds(L, L)] = tidx[ib][pl.ds(off + L, L)]
                compute(b)
                pltpu.async_copy(wv[b], numer.at[tb], ss[b], add=True)
                pltpu.async_copy(wt[b], denom.at[tb], ss[b], add=True)

              @pl.when(c + 2 < hi)
              def _():
                issue_gathers(b, ib, off + 2 * C)
            return carry2

          lax.fori_loop(0, BC // 2, pair_body, 0, unroll=False)
      return carry

    lax.fori_loop(0, (nb + 1) // 2, block_pair, 0, unroll=False)

    for b in range(2):
      wait_scatters(b)
    plsc.subcore_barrier()

    pltpu.sync_copy(numer.at[pl.ds(row0, rows_per_sub), :],
                    onum_hbm.at[cid, pl.ds(row0, rows_per_sub), :])
    pltpu.sync_copy(denom.at[pl.ds(row0, rows_per_sub), :],
                    oden_hbm.at[cid, pl.ds(row0, rows_per_sub), :])

  return edge_kernel


def _out_body(num_ref, den_ref, x_ref, wa_ref, ba_ref, r16_ref, mix_ref,
              o_ref):
  num = num_ref[0] + num_ref[1]
  den = den_ref[0] + den_ref[1]
  den_rep = den @ r16_ref[...]
  m = num / (den_rep + 1e-16)
  out = m @ wa_ref[...] + ba_ref[...]
  a = mix_ref[0, 0]
  o_ref[...] = a * out + (1.0 - a) * x_ref[...]


def _combine(num2, den2, x, Wa, ba, r16, mix, blk_n):
  n = x.shape[0]
  grid = (n // blk_n,)
  return pl.pallas_call(
      _out_body,
      grid=grid,
      in_specs=[
          pl.BlockSpec((NC, blk_n, HD), lambda i: (0, i, 0)),
          pl.BlockSpec((NC, blk_n, L), lambda i: (0, i, 0)),
          pl.BlockSpec((blk_n, HD), lambda i: (i, 0)),
          pl.BlockSpec((HD, HD), lambda i: (0, 0)),
          pl.BlockSpec((1, HD), lambda i: (0, 0)),
          pl.BlockSpec((L, HD), lambda i: (0, 0)),
          pl.BlockSpec((1, 1), lambda i: (0, 0)),
      ],
      out_specs=pl.BlockSpec((blk_n, HD), lambda i: (i, 0)),
      out_shape=jax.ShapeDtypeStruct((n, HD), jnp.float32),
  )(num2, den2, x, Wa, ba.reshape(1, HD), r16, mix)


def kernel(x, edge_index, Wq, bq, Wk, bk, Wv, bv, Wa, ba, skip, a_rel,
           m_rel, p_rel):
  n = x.shape[0]
  e = edge_index.shape[1]

  eyeh = jnp.eye(H, dtype=jnp.float32)
  a_blk = (eyeh[:, None, :, None] * a_rel[:, :, None, :]).reshape(HD, HD)
  m_blk = (eyeh[:, None, :, None] * m_rel[:, :, None, :]).reshape(HD, HD)
  prow = jnp.repeat(p_rel / jnp.sqrt(jnp.float32(D)), D).reshape(1, HD)
  r16 = (jnp.eye(L, H, dtype=jnp.float32)[:, :, None] *
         jnp.ones((D,), jnp.float32)).reshape(L, HD)
  mix = jax.nn.sigmoid(skip).reshape(1, 1).astype(jnp.float32)

  q, kv = _project(x, Wq, bq, Wk, bk, Wv, bv, a_blk, m_blk, prow, 1000)

  zpad = jnp.zeros((IDXN,), jnp.int32)
  srcp = jnp.concatenate([edge_index[0], zpad])
  tgtp = jnp.concatenate([edge_index[1], zpad])

  n_pad = -(-n // (NS * C)) * (NS * C)
  edge_kernel = _make_edge_kernel(n_pad, e)
  num2, den2 = edge_kernel(q, kv, srcp, tgtp)

  return _combine(num2, den2, x, Wa, ba, r16, mix, 1000)
